# baseline (speedup 1.0000x reference)
import functools

import jax
import jax.numpy as jnp
from jax import lax
from jax.experimental import pallas as pl
from jax.experimental.pallas import tpu as pltpu
from jax.experimental.pallas import tpu_sc as plsc

B = 16384
CONT = 13
NF = 26
V = 100000
D = 16

NC = 2
NS = 16
NW = NC * NS
ROWS_W = B // NW
RP = B // 128
WP = ROWS_W // 128
CB = 128
NCHUNK = ROWS_W // CB
NFH = NF // 2
FDH = NFH * D
OUT_W = CONT + NF * D


def _gather_half(row_off, cate_hbm, cont_hbm, table_hbm, out_hbm,
                 fcate_v, gbuf_v, gsem, wsem, csem):
    wid = lax.axis_index("s") * NC + lax.axis_index("c")
    base = wid * ROWS_W

    if cont_hbm is not None:
        for k in range(CONT):
            pltpu.async_copy(cont_hbm.at[k, pl.ds(base, ROWS_W)],
                             out_hbm.at[k, pl.ds(base, ROWS_W)], csem)

    idescs = []
    for f in range(NFH):
        idescs.append(pltpu.async_copy(
            cate_hbm.at[pl.ds(f * RP + wid * WP, WP)], fcate_v.at[f], wsem))
    for dsc in idescs:
        dsc.wait()

    def fire(c, buf):
        def g_body(f, carry):
            for d in range(D):
                pltpu.async_copy(
                    table_hbm.at[f * D + d].at[fcate_v.at[f, c]],
                    gbuf_v.at[buf, f * D + d], gsem)
            return carry
        lax.fori_loop(0, NFH, g_body, 0)

    def drain(c, buf):
        def g_body(f, carry):
            for d in range(D):
                pltpu.make_async_copy(
                    table_hbm.at[f * D + d].at[fcate_v.at[f, c]],
                    gbuf_v.at[buf, f * D + d], gsem).wait()
            return carry
        lax.fori_loop(0, NFH, g_body, 0)

    fire(0, 0)

    def chunk_body(c, carry):
        buf = lax.rem(c, 2)
        drain(c, buf)

        @pl.when(c + 1 < NCHUNK)
        def _():
            fire(c + 1, lax.rem(c + 1, 2))

        def w_body(f, carry2):
            for d in range(D):
                pltpu.async_copy(
                    gbuf_v.at[buf, f * D + d],
                    out_hbm.at[row_off + f * D + d,
                               pl.ds(base + c * CB, CB)], wsem)
            return carry2

        lax.fori_loop(0, NFH, w_body, 0)

        def wd_body(f, carry2):
            for d in range(D):
                pltpu.make_async_copy(
                    gbuf_v.at[buf, f * D + d],
                    out_hbm.at[row_off + f * D + d,
                               pl.ds(base + c * CB, CB)], wsem).wait()
            return carry2

        lax.fori_loop(0, NFH, wd_body, 0)
        return carry

    lax.fori_loop(0, NCHUNK, chunk_body, 0)

    if cont_hbm is not None:
        for k in range(CONT):
            pltpu.make_async_copy(
                cont_hbm.at[k, pl.ds(base, ROWS_W)],
                out_hbm.at[k, pl.ds(base, ROWS_W)], csem).wait()


def _half_a(cate_hbm, cont_hbm, table_hbm, out_hbm,
            fcate_v, gbuf_v, gsem, wsem, csem):
    _gather_half(CONT, cate_hbm, cont_hbm, table_hbm, out_hbm,
                 fcate_v, gbuf_v, gsem, wsem, csem)


def _half_b(cate_hbm, table_hbm, out_hbm,
            fcate_v, gbuf_v, gsem, wsem, csem):
    _gather_half(0, cate_hbm, None, table_hbm, out_hbm,
                 fcate_v, gbuf_v, gsem, wsem, csem)


@jax.jit
def kernel(x_cont, x_cate, tables):
    cate_t = x_cate.T
    cont_t = x_cont.T
    table_a = tables[:NFH].transpose(0, 2, 1).reshape(NFH * D, V)
    table_b = tables[NFH:].transpose(0, 2, 1).reshape(NFH * D, V)
    cate_a = cate_t[:NFH].reshape(NFH * RP, 128)
    cate_b = cate_t[NFH:].reshape(NFH * RP, 128)
    mesh = plsc.VectorSubcoreMesh(core_axis_name="c", subcore_axis_name="s")
    scratch = [
        pltpu.VMEM((NFH, WP, 128), jnp.int32),
        pltpu.VMEM((2, FDH, CB), jnp.float32),
        pltpu.SemaphoreType.DMA,
        pltpu.SemaphoreType.DMA,
        pltpu.SemaphoreType.DMA,
    ]
    run_a = functools.partial(
        pl.kernel, mesh=mesh,
        compiler_params=pltpu.CompilerParams(use_tc_tiling_on_sc=False),
        out_type=jax.ShapeDtypeStruct((CONT + FDH, B), jnp.float32),
        scratch_types=scratch)(_half_a)
    run_b = functools.partial(
        pl.kernel, mesh=mesh,
        compiler_params=pltpu.CompilerParams(use_tc_tiling_on_sc=False),
        out_type=jax.ShapeDtypeStruct((FDH, B), jnp.float32),
        scratch_types=scratch)(_half_b)
    yt_a = run_a(cate_a, cont_t, table_a)
    yt_b = run_b(cate_b, table_b)
    return jnp.concatenate([yt_a, yt_b], axis=0).T


# speedup vs baseline: 1.0550x; 1.0550x over previous
import functools

import jax
import jax.numpy as jnp
from jax import lax
from jax.experimental import pallas as pl
from jax.experimental.pallas import tpu as pltpu
from jax.experimental.pallas import tpu_sc as plsc

B = 16384
CONT = 13
NF = 26
V = 100000
D = 16

NC = 2
NS = 16
NW = NC * NS
ROWS_W = B // NW
RP = B // 128
WP = ROWS_W // 128
CB = 128
NCHUNK = ROWS_W // CB
FD = NF * D
OUT_W = CONT + FD


def _sc_kernel(cate_hbm, cont_hbm, table_hbm, out_hbm,
               fcate_v, gbuf_v, gsem, wsem, csem):
    wid = lax.axis_index("s") * NC + lax.axis_index("c")
    base = wid * ROWS_W

    for k in range(CONT):
        pltpu.async_copy(cont_hbm.at[k, pl.ds(base, ROWS_W)],
                         out_hbm.at[k, pl.ds(base, ROWS_W)], csem)

    idescs = []
    for f in range(NF):
        idescs.append(pltpu.async_copy(
            cate_hbm.at[pl.ds(f * RP + wid * WP, WP)], fcate_v.at[f], wsem))
    for dsc in idescs:
        dsc.wait()

    def fire(c, buf):
        def g_body(f, carry):
            for d in range(D):
                pltpu.async_copy(
                    table_hbm.at[f * D + d].at[fcate_v.at[f, c]],
                    gbuf_v.at[buf, f * D + d], gsem)
            return carry
        lax.fori_loop(0, NF, g_body, 0)

    def drain(c, buf):
        def g_body(f, carry):
            for d in range(D):
                pltpu.make_async_copy(
                    table_hbm.at[f * D + d].at[fcate_v.at[f, c]],
                    gbuf_v.at[buf, f * D + d], gsem).wait()
            return carry
        lax.fori_loop(0, NF, g_body, 0)

    fire(0, 0)

    def chunk_body(c, carry):
        buf = lax.rem(c, 2)
        drain(c, buf)

        @pl.when(c + 1 < NCHUNK)
        def _():
            fire(c + 1, lax.rem(c + 1, 2))

        def w_body(f, carry2):
            for d in range(D):
                pltpu.async_copy(
                    gbuf_v.at[buf, f * D + d],
                    out_hbm.at[CONT + f * D + d, pl.ds(base + c * CB, CB)],
                    wsem)
            return carry2

        lax.fori_loop(0, NF, w_body, 0)

        def wd_body(f, carry2):
            for d in range(D):
                pltpu.make_async_copy(
                    gbuf_v.at[buf, f * D + d],
                    out_hbm.at[CONT + f * D + d, pl.ds(base + c * CB, CB)],
                    wsem).wait()
            return carry2

        lax.fori_loop(0, NF, wd_body, 0)
        return carry

    lax.fori_loop(0, NCHUNK, chunk_body, 0)

    for k in range(CONT):
        pltpu.make_async_copy(cont_hbm.at[k, pl.ds(base, ROWS_W)],
                              out_hbm.at[k, pl.ds(base, ROWS_W)], csem).wait()


@jax.jit
def kernel(x_cont, x_cate, tables):
    cate_t = x_cate.T.reshape(NF * RP, 128)
    cont_t = x_cont.T
    table_t = tables.transpose(0, 2, 1).reshape(NF * D, V)
    mesh = plsc.VectorSubcoreMesh(core_axis_name="c", subcore_axis_name="s")
    run = functools.partial(
        pl.kernel,
        mesh=mesh,
        compiler_params=pltpu.CompilerParams(use_tc_tiling_on_sc=False),
        out_type=jax.ShapeDtypeStruct((OUT_W, B), jnp.float32),
        scratch_types=[
            pltpu.VMEM((NF, WP, 128), jnp.int32),
            pltpu.VMEM((2, FD, CB), jnp.float32),
            pltpu.SemaphoreType.DMA,
            pltpu.SemaphoreType.DMA,
            pltpu.SemaphoreType.DMA,
        ],
    )(_sc_kernel)
    yt = run(cate_t, cont_t, table_t)
    return yt.T
